# baseline (speedup 1.0000x reference)
import functools

import jax
import jax.numpy as jnp
from jax import lax
from jax.experimental import pallas as pl
from jax.experimental.pallas import tpu as pltpu
from jax.experimental.pallas import tpu_sc as plsc

_NC = 2
_NS = 16
_NW = _NC * _NS
_EPAD = 128
_SPAD = 24


def _tc_tables(tok_pad, pos_pad, w_pad, b2):
    vocab_in, _ = tok_pad.shape
    seq = pos_pad.shape[0]
    vocab = w_pad.shape[1]

    def body(tok_ref, pos_ref, w_ref, b_ref, tl3_ref, p2_ref):
        w = w_ref[...]
        y = (jnp.dot(tok_ref[...], w, preferred_element_type=jnp.float32)
             + b_ref[...])
        tl3_ref[...] = y.reshape(vocab_in, vocab // 128, 128)
        p2_ref[...] = jnp.dot(
            pos_ref[...], w, preferred_element_type=jnp.float32)

    return pl.pallas_call(
        body,
        out_shape=(
            jax.ShapeDtypeStruct((vocab_in, vocab // 128, 128), jnp.float32),
            jax.ShapeDtypeStruct((seq, vocab), jnp.float32),
        ),
    )(tok_pad, pos_pad, w_pad, b2)


def _sc_emit(x3, tl2, p2, batch, seq, vocab, vpad):
    npb = batch // _NW
    ngrp = npb // 2
    last = vocab - 16
    mesh = plsc.VectorSubcoreMesh(
        core_axis_name="c", subcore_axis_name="s",
        num_cores=_NC, num_subcores=_NS,
    )

    @functools.partial(
        pl.kernel,
        out_type=jax.ShapeDtypeStruct((batch, seq, vocab), jnp.float32),
        mesh=mesh,
        compiler_params=pltpu.CompilerParams(needs_layout_passes=False),
        scratch_types=[
            pltpu.VMEM((npb * _SPAD,), jnp.int32),
            pltpu.VMEM((seq, vpad), jnp.float32),
            pltpu.VMEM((2, _SPAD, vpad // 128, 128), jnp.float32),
            pltpu.VMEM((2, seq, vocab), jnp.float32),
            [pltpu.SemaphoreType.DMA] * 2,
            [pltpu.SemaphoreType.DMA] * 2,
        ],
    )
    def body(x_hbm, tl2_hbm, p2_hbm, out_hbm, idx_v, p2_v, raw_v, cnv_v,
             gsem, osem):
        wid = lax.axis_index("s") * _NC + lax.axis_index("c")
        base = wid * npb
        pltpu.sync_copy(x_hbm.at[wid], idx_v)
        pltpu.sync_copy(p2_hbm, p2_v)
        for k in range(2):
            pltpu.async_copy(
                tl2_hbm.at[idx_v.at[pl.ds(k * _SPAD, _SPAD)]],
                raw_v.at[k], gsem[k])

        def convert(k):
            def col(i, carry):
                t = i // 8
                c2 = pl.multiple_of((i % 8) * 16, 16)
                c = pl.multiple_of(i * 16, 16)
                for r in range(seq):
                    cnv_v[k, r, pl.ds(c, 16)] = (
                        raw_v[k, r, t, pl.ds(c2, 16)]
                        + p2_v[r, pl.ds(c, 16)])
                return carry
            lax.fori_loop(0, vocab // 16, col, 0)
            if vocab % 16:
                c0 = (vocab // 16) * 16
                nv = vocab - c0
                lanes = lax.iota(jnp.int32, 16)
                mask = lanes < nv
                kv = jnp.full((16,), k, jnp.int32)
                cols = c0 + lanes
                for r in range(seq):
                    v = (raw_v[k, r, c0 // 128, pl.ds(c0 % 128, 16)]
                         + p2_v[r, pl.ds(c0, 16)])
                    plsc.store_scatter(
                        cnv_v, [kv, jnp.full((16,), r, jnp.int32), cols],
                        v, mask=mask)

        def group(g, carry):
            for k in range(2):
                b = g * 2 + k
                pltpu.make_async_copy(
                    tl2_hbm.at[idx_v.at[pl.ds(b * _SPAD, _SPAD)]],
                    raw_v.at[k], gsem[k]).wait()

                @pl.when(g > 0)
                def _():
                    pltpu.make_async_copy(
                        cnv_v.at[k], out_hbm.at[base + b - 2],
                        osem[k]).wait()

                convert(k)
                pltpu.async_copy(cnv_v.at[k], out_hbm.at[base + b], osem[k])

                @pl.when(b + 2 < npb)
                def _():
                    pltpu.async_copy(
                        tl2_hbm.at[idx_v.at[pl.ds((b + 2) * _SPAD, _SPAD)]],
                        raw_v.at[k], gsem[k])
            return carry

        lax.fori_loop(0, ngrp, group, 0)
        for k in range(2):
            pltpu.make_async_copy(
                cnv_v.at[k], out_hbm.at[base + npb - 2 + k], osem[k]).wait()

    return body(x3, tl2, p2)


def kernel(X, token_table, pos_table, W, b):
    batch, seq = X.shape
    vocab_in, emb = token_table.shape
    vocab = W.shape[1]

    vpad = 1024
    tok_pad = jnp.pad(token_table, ((0, 0), (0, _EPAD - emb)))
    pos_pad = jnp.pad(pos_table, ((0, 0), (0, _EPAD - emb)))
    w_pad = jnp.pad(W, ((0, _EPAD - emb), (0, vpad - vocab)))
    b2 = jnp.pad(b, (0, vpad - vocab)).reshape(1, vpad)
    tl2, p2 = _tc_tables(tok_pad, pos_pad, w_pad, b2)

    xp = jnp.pad(X.astype(jnp.int32), ((0, 0), (0, _SPAD - seq)))
    x3 = xp.reshape(_NW, (batch // _NW) * _SPAD)
    return _sc_emit(x3, tl2, p2, batch, seq, vocab, vpad)


# speedup vs baseline: 1.4439x; 1.4439x over previous
import functools

import jax
import jax.numpy as jnp
from jax import lax
from jax.experimental import pallas as pl
from jax.experimental.pallas import tpu as pltpu
from jax.experimental.pallas import tpu_sc as plsc

_NC = 2
_NS = 16
_NW = _NC * _NS
_EPAD = 128
_SPAD = 24


def _tc_tables(tok_pad, pos_pad, w_pad, b2):
    vocab_in, _ = tok_pad.shape
    seq = pos_pad.shape[0]
    vocab = w_pad.shape[1]

    def body(tok_ref, pos_ref, w_ref, b_ref, tl3_ref, p2_ref):
        w = w_ref[...]
        y = (jnp.dot(tok_ref[...], w, preferred_element_type=jnp.float32)
             + b_ref[...])
        tl3_ref[...] = y.reshape(vocab_in, vocab // 128, 128)
        p2_ref[...] = jnp.dot(
            pos_ref[...], w, preferred_element_type=jnp.float32)

    return pl.pallas_call(
        body,
        out_shape=(
            jax.ShapeDtypeStruct((vocab_in, vocab // 128, 128), jnp.float32),
            jax.ShapeDtypeStruct((seq, vocab), jnp.float32),
        ),
    )(tok_pad, pos_pad, w_pad, b2)


def _sc_emit(x3, tl2, p2, batch, seq, vocab, vpad):
    npb = batch // _NW
    ngrp = npb // 2
    last = vocab - 16
    mesh = plsc.VectorSubcoreMesh(
        core_axis_name="c", subcore_axis_name="s",
        num_cores=_NC, num_subcores=_NS,
    )

    @functools.partial(
        pl.kernel,
        out_type=jax.ShapeDtypeStruct((batch, seq, vocab), jnp.float32),
        mesh=mesh,
        compiler_params=pltpu.CompilerParams(needs_layout_passes=False),
        scratch_types=[
            pltpu.VMEM((npb * _SPAD,), jnp.int32),
            pltpu.VMEM((seq, vpad), jnp.float32),
            pltpu.VMEM((2, _SPAD, vpad // 128, 128), jnp.float32),
            pltpu.VMEM((2, seq, vocab), jnp.float32),
            [pltpu.SemaphoreType.DMA] * 2,
            [pltpu.SemaphoreType.DMA] * 2,
        ],
    )
    def body(x_hbm, tl2_hbm, p2_hbm, out_hbm, idx_v, p2_v, raw_v, cnv_v,
             gsem, osem):
        wid = lax.axis_index("s") * _NC + lax.axis_index("c")
        base = wid * npb
        pltpu.sync_copy(x_hbm.at[wid], idx_v)
        pltpu.sync_copy(p2_hbm, p2_v)
        for k in range(2):
            pltpu.async_copy(
                tl2_hbm.at[idx_v.at[pl.ds(k * _SPAD, _SPAD)]],
                raw_v.at[k], gsem[k])

        def convert(k):
            def col(i, carry):
                t = i // 8
                c2 = pl.multiple_of((i % 8) * 16, 16)
                c = pl.multiple_of(i * 16, 16)
                for r in range(seq):
                    cnv_v[k, r, pl.ds(c, 16)] = (
                        raw_v[k, r, t, pl.ds(c2, 16)]
                        + p2_v[r, pl.ds(c, 16)])
                return carry
            lax.fori_loop(0, vocab // 16, col, 0)
            if vocab % 16:
                c0 = (vocab // 16) * 16
                nv = vocab - c0
                lanes = lax.iota(jnp.int32, 16)
                mask = lanes < nv
                kv = jnp.full((16,), k, jnp.int32)
                cols = c0 + lanes
                for r in range(seq):
                    v = (raw_v[k, r, c0 // 128, pl.ds(c0 % 128, 16)]
                         + p2_v[r, pl.ds(c0, 16)])
                    plsc.store_scatter(
                        cnv_v, [kv, jnp.full((16,), r, jnp.int32), cols],
                        v, mask=mask)

        def group(g, carry):
            for k in range(2):
                b = g * 2 + k
                pltpu.make_async_copy(
                    tl2_hbm.at[idx_v.at[pl.ds(b * _SPAD, _SPAD)]],
                    raw_v.at[k], gsem[k]).wait()

                convert(k)

                @pl.when(b + 2 < npb)
                def _():
                    pltpu.async_copy(
                        tl2_hbm.at[idx_v.at[pl.ds((b + 2) * _SPAD, _SPAD)]],
                        raw_v.at[k], gsem[k])
            return carry

        lax.fori_loop(0, ngrp, group, 0)
        for k in range(2):
            pltpu.async_copy(cnv_v.at[k], out_hbm.at[base + npb - 2 + k],
                             osem[k]).wait()

    return body(x3, tl2, p2)


def kernel(X, token_table, pos_table, W, b):
    batch, seq = X.shape
    vocab_in, emb = token_table.shape
    vocab = W.shape[1]

    vpad = 1024
    tok_pad = jnp.pad(token_table, ((0, 0), (0, _EPAD - emb)))
    pos_pad = jnp.pad(pos_table, ((0, 0), (0, _EPAD - emb)))
    w_pad = jnp.pad(W, ((0, _EPAD - emb), (0, vpad - vocab)))
    b2 = jnp.pad(b, (0, vpad - vocab)).reshape(1, vpad)
    tl2, p2 = _tc_tables(tok_pad, pos_pad, w_pad, b2)

    xp = jnp.pad(X.astype(jnp.int32), ((0, 0), (0, _SPAD - seq)))
    x3 = xp.reshape(_NW, (batch // _NW) * _SPAD)
    return _sc_emit(x3, tl2, p2, batch, seq, vocab, vpad)


# speedup vs baseline: 3.1419x; 2.1759x over previous
import functools

import jax
import jax.numpy as jnp
from jax import lax
from jax.experimental import pallas as pl
from jax.experimental.pallas import tpu as pltpu
from jax.experimental.pallas import tpu_sc as plsc

_NC = 2
_NS = 16
_NW = _NC * _NS
_CHUNK = 128
_EPAD = 128


def _sc_gather(x_grouped, table_pad, n_rows):
    per_w = n_rows // _NW
    nchunk = per_w // _CHUNK
    mesh = plsc.VectorSubcoreMesh(
        core_axis_name="c", subcore_axis_name="s",
        num_cores=_NC, num_subcores=_NS,
    )

    @functools.partial(
        pl.kernel,
        out_type=jax.ShapeDtypeStruct((n_rows, _EPAD), jnp.float32),
        mesh=mesh,
        scratch_types=[
            pltpu.VMEM((nchunk, _CHUNK), jnp.int32),
            pltpu.VMEM((2, _CHUNK, _EPAD), jnp.float32),
            pltpu.SemaphoreType.DMA,
            pltpu.SemaphoreType.DMA,
        ],
    )
    def gather_kernel(x_hbm, table_hbm, out_hbm, idx_v, rows_v, sem0, sem1):
        wid = lax.axis_index("s") * _NC + lax.axis_index("c")
        base = wid * per_w
        pltpu.sync_copy(x_hbm.at[wid], idx_v)
        sems = (sem0, sem1)
        copies = [None, None]
        copies[0] = pltpu.async_copy(
            table_hbm.at[idx_v.at[0]], rows_v.at[0], sems[0])
        for j in range(nchunk):
            cur = j % 2
            if j + 1 < nchunk:
                nxt = (j + 1) % 2
                copies[nxt] = pltpu.async_copy(
                    table_hbm.at[idx_v.at[j + 1]], rows_v.at[nxt], sems[nxt])
            copies[cur].wait()
            pltpu.sync_copy(
                rows_v.at[cur], out_hbm.at[pl.ds(base + j * _CHUNK, _CHUNK)])

    return gather_kernel(x_grouped, table_pad)


def _tc_linear(h2, pos_rep, w_pad, b2, batch, seq, b_blk):
    vocab = w_pad.shape[1]
    r_blk = b_blk * seq
    grid = (batch // b_blk,)

    def body(h_ref, pos_ref, w_ref, b_ref, out_ref):
        x = h_ref[...] + pos_ref[...]
        y = jnp.dot(x, w_ref[...], preferred_element_type=jnp.float32)
        out_ref[...] = y.reshape(b_blk, seq, vocab) + b_ref[...]

    return pl.pallas_call(
        body,
        grid=grid,
        in_specs=[
            pl.BlockSpec((r_blk, _EPAD), lambda i: (i, 0)),
            pl.BlockSpec((r_blk, _EPAD), lambda i: (0, 0)),
            pl.BlockSpec((_EPAD, vocab), lambda i: (0, 0)),
            pl.BlockSpec((1, 1, vocab), lambda i: (0, 0, 0)),
        ],
        out_specs=pl.BlockSpec((b_blk, seq, vocab), lambda i: (i, 0, 0)),
        out_shape=jax.ShapeDtypeStruct((batch, seq, vocab), jnp.float32),
        compiler_params=pltpu.CompilerParams(
            dimension_semantics=("parallel",)),
    )(h2, pos_rep, w_pad, b2)


def kernel(X, token_table, pos_table, W, b):
    batch, seq = X.shape
    vocab, emb = token_table.shape
    vocab_out = W.shape[1]
    n_rows = batch * seq
    per_w = n_rows // _NW

    x_grouped = X.astype(jnp.int32).reshape(_NW, per_w // _CHUNK, _CHUNK)
    table_pad = jnp.pad(token_table, ((0, 0), (0, _EPAD - emb)))
    h2 = _sc_gather(x_grouped, table_pad, n_rows)

    b_blk = 64
    pos_rep = jnp.tile(jnp.pad(pos_table, ((0, 0), (0, _EPAD - emb))),
                       (b_blk, 1))
    w_pad = jnp.pad(W, ((0, _EPAD - emb), (0, 0)))
    b3 = b.reshape(1, 1, vocab_out)
    return _tc_linear(h2, pos_rep, w_pad, b3, batch, seq, b_blk)


# speedup vs baseline: 3.1823x; 1.0128x over previous
import functools

import jax
import jax.numpy as jnp
from jax import lax
from jax.experimental import pallas as pl
from jax.experimental.pallas import tpu as pltpu
from jax.experimental.pallas import tpu_sc as plsc

_NC = 2
_NS = 16
_NW = _NC * _NS
_CHUNK = 128
_EPAD = 128


def _sc_gather(x_grouped, table_pad, n_rows):
    per_w = n_rows // _NW
    nchunk = per_w // _CHUNK
    mesh = plsc.VectorSubcoreMesh(
        core_axis_name="c", subcore_axis_name="s",
        num_cores=_NC, num_subcores=_NS,
    )

    @functools.partial(
        pl.kernel,
        out_type=jax.ShapeDtypeStruct((n_rows, _EPAD), jnp.float32),
        mesh=mesh,
        scratch_types=[
            pltpu.VMEM((nchunk, _CHUNK), jnp.int32),
            pltpu.VMEM((2, _CHUNK, _EPAD), jnp.float32),
            pltpu.SemaphoreType.DMA,
            pltpu.SemaphoreType.DMA,
        ],
    )
    def gather_kernel(x_hbm, table_hbm, out_hbm, idx_v, rows_v, sem0, sem1):
        wid = lax.axis_index("s") * _NC + lax.axis_index("c")
        base = wid * per_w
        pltpu.sync_copy(x_hbm.at[wid], idx_v)
        sems = (sem0, sem1)
        copies = [None, None]
        copies[0] = pltpu.async_copy(
            table_hbm.at[idx_v.at[0]], rows_v.at[0], sems[0])
        for j in range(nchunk):
            cur = j % 2
            if j + 1 < nchunk:
                nxt = (j + 1) % 2
                copies[nxt] = pltpu.async_copy(
                    table_hbm.at[idx_v.at[j + 1]], rows_v.at[nxt], sems[nxt])
            copies[cur].wait()
            pltpu.sync_copy(
                rows_v.at[cur], out_hbm.at[pl.ds(base + j * _CHUNK, _CHUNK)])

    return gather_kernel(x_grouped, table_pad)


def _tc_linear(h2, pos_rep, w_pad, b2, batch, seq, b_blk):
    vocab = w_pad.shape[1]
    r_blk = b_blk * seq
    grid = (batch // b_blk,)

    def body(h_ref, pos_ref, w_ref, b_ref, out_ref):
        x = h_ref[...] + pos_ref[...]
        y = jnp.dot(x, w_ref[...], preferred_element_type=jnp.float32)
        out_ref[...] = y.reshape(b_blk, seq, vocab) + b_ref[...]

    return pl.pallas_call(
        body,
        grid=grid,
        in_specs=[
            pl.BlockSpec((r_blk, _EPAD), lambda i: (i, 0)),
            pl.BlockSpec((r_blk, _EPAD), lambda i: (0, 0)),
            pl.BlockSpec((_EPAD, vocab), lambda i: (0, 0)),
            pl.BlockSpec((1, 1, vocab), lambda i: (0, 0, 0)),
        ],
        out_specs=pl.BlockSpec((b_blk, seq, vocab), lambda i: (i, 0, 0)),
        out_shape=jax.ShapeDtypeStruct((batch, seq, vocab), jnp.float32),
        compiler_params=pltpu.CompilerParams(
            dimension_semantics=("parallel",)),
    )(h2, pos_rep, w_pad, b2)


def kernel(X, token_table, pos_table, W, b):
    batch, seq = X.shape
    vocab, emb = token_table.shape
    vocab_out = W.shape[1]
    n_rows = batch * seq
    per_w = n_rows // _NW

    x_grouped = X.astype(jnp.int32).reshape(_NW, per_w // _CHUNK, _CHUNK)
    table_pad = jnp.pad(token_table, ((0, 0), (0, _EPAD - emb)))
    h2 = _sc_gather(x_grouped, table_pad, n_rows)

    b_blk = 128
    pos_rep = jnp.tile(jnp.pad(pos_table, ((0, 0), (0, _EPAD - emb))),
                       (b_blk, 1))
    w_pad = jnp.pad(W, ((0, _EPAD - emb), (0, 0)))
    b3 = b.reshape(1, 1, vocab_out)
    return _tc_linear(h2, pos_rep, w_pad, b3, batch, seq, b_blk)
